# baseline (speedup 1.0000x reference)
import jax
import jax.numpy as jnp
from jax.experimental import pallas as pl

N = 16384
D = 256
K = 1024
BN = 512


def _assign_kernel(x_ref, c_ref, out_ref):
    x = x_ref[...]
    c = c_ref[...]
    prod = jnp.dot(x, c, preferred_element_type=jnp.float32)
    c_norm = jnp.sum(c * c, axis=0, keepdims=True)
    scores = c_norm - 2.0 * prod
    m = jnp.min(scores, axis=-1, keepdims=True)
    idx = jax.lax.broadcasted_iota(jnp.int32, scores.shape, 1)
    am = jnp.min(jnp.where(scores == m, idx, K), axis=-1)
    out_ref[...] = am.reshape(1, 1, BN)


def kernel(x, centroids):
    out = pl.pallas_call(
        _assign_kernel,
        grid=(N // BN,),
        in_specs=[
            pl.BlockSpec((BN, D), lambda i: (i, 0)),
            pl.BlockSpec((D, K), lambda i: (0, 0)),
        ],
        out_specs=pl.BlockSpec((1, 1, BN), lambda i: (i, 0, 0)),
        out_shape=jax.ShapeDtypeStruct((N // BN, 1, BN), jnp.int32),
    )(x, centroids)
    return out.reshape(N)
